# Initial kernel scaffold; baseline (speedup 1.0000x reference)
import jax
import jax.numpy as jnp
from jax.experimental import pallas as pl


def kernel(batch, W_opcode, W_operand1, W_operand2, W_pos):
    raise NotImplementedError("write your pallas kernel here")


# speedup vs baseline: 4.8656x; 4.8656x over previous
import functools

import jax
import jax.numpy as jnp
from jax import lax
from jax.experimental import pallas as pl
from jax.experimental.pallas import tpu as pltpu
from jax.experimental.pallas import tpu_sc as plsc

NUM_CORES = 2
NUM_SUBCORES = 16
NUM_WORKERS = NUM_CORES * NUM_SUBCORES
LANES = 16
CHUNK = 128


def _make_kernel(n_tok, L, D, V):
    chunks_per_worker = n_tok // (NUM_WORKERS * CHUNK)
    DM = 3 * D
    mesh = plsc.VectorSubcoreMesh(core_axis_name="c", subcore_axis_name="s")

    @functools.partial(
        pl.kernel,
        mesh=mesh,
        compiler_params=pltpu.CompilerParams(use_tc_tiling_on_sc=False),
        out_type=jax.ShapeDtypeStruct((n_tok, DM), jnp.float32),
        scratch_types=[
            pltpu.VMEM((CHUNK,), jnp.int32),
            pltpu.VMEM((CHUNK,), jnp.int32),
            pltpu.VMEM((CHUNK,), jnp.int32),
            pltpu.VMEM((CHUNK, D), jnp.float32),
            pltpu.VMEM((CHUNK, D), jnp.float32),
            pltpu.VMEM((CHUNK, D), jnp.float32),
            pltpu.VMEM((L, D), jnp.float32),
            pltpu.VMEM((L, D), jnp.float32),
            pltpu.VMEM((L, D), jnp.float32),
            pltpu.SemaphoreType.DMA,
        ],
    )
    def emb_kernel(idx0, idx1, idx2, t0, t1, t2, p0, p1, p2, out,
                   iv0, iv1, iv2, r0, r1, r2, pv0, pv1, pv2, sem):
        wid = lax.axis_index("s") * NUM_CORES + lax.axis_index("c")

        pltpu.sync_copy(p0, pv0)
        pltpu.sync_copy(p1, pv1)
        pltpu.sync_copy(p2, pv2)

        def chunk_body(c, carry):
            base = (wid * chunks_per_worker + c) * CHUNK

            pltpu.sync_copy(idx0.at[pl.ds(base, CHUNK)], iv0)
            pltpu.sync_copy(idx1.at[pl.ds(base, CHUNK)], iv1)
            pltpu.sync_copy(idx2.at[pl.ds(base, CHUNK)], iv2)

            cp0 = pltpu.async_copy(t0.at[iv0], r0, sem)
            cp1 = pltpu.async_copy(t1.at[iv1], r1, sem)
            cp2 = pltpu.async_copy(t2.at[iv2], r2, sem)
            cp0.wait()
            cp1.wait()
            cp2.wait()

            def tok_body(t, carry2):
                pr = lax.rem(base + t, L)
                for r, pv in ((r0, pv0), (r1, pv1), (r2, pv2)):
                    for h in range(D // LANES):
                        x = pv[pr, pl.ds(h * LANES, LANES)]
                        plsc.addupdate(r.at[t, pl.ds(h * LANES, LANES)], x)
                return carry2

            lax.fori_loop(0, CHUNK, tok_body, 0)

            pltpu.sync_copy(r0, out.at[pl.ds(base, CHUNK), pl.ds(0, D)])
            pltpu.sync_copy(r1, out.at[pl.ds(base, CHUNK), pl.ds(D, D)])
            pltpu.sync_copy(r2, out.at[pl.ds(base, CHUNK), pl.ds(2 * D, D)])
            return carry

        lax.fori_loop(0, chunks_per_worker, chunk_body, 0)

    return emb_kernel


def kernel(batch, W_opcode, W_operand1, W_operand2, W_pos):
    B, L, _ = batch.shape
    V, D = W_opcode.shape
    n_tok = B * L
    assert n_tok % (NUM_WORKERS * CHUNK) == 0

    idx = batch.astype(jnp.int32)
    idx0 = idx[:, :, 0].reshape(n_tok)
    idx1 = idx[:, :, 1].reshape(n_tok)
    idx2 = idx[:, :, 2].reshape(n_tok)
    pos0 = jnp.asarray(W_pos[:L, 0:D])
    pos1 = jnp.asarray(W_pos[:L, D:2 * D])
    pos2 = jnp.asarray(W_pos[:L, 2 * D:3 * D])

    out = _make_kernel(n_tok, L, D, V)(
        idx0, idx1, idx2, W_opcode, W_operand1, W_operand2, pos0, pos1, pos2)
    return out.reshape(B, L, 3 * D)


# speedup vs baseline: 5.2610x; 1.0812x over previous
import functools

import jax
import jax.numpy as jnp
from jax import lax
from jax.experimental import pallas as pl
from jax.experimental.pallas import tpu as pltpu
from jax.experimental.pallas import tpu_sc as plsc

NUM_CORES = 2
NUM_SUBCORES = 16
NUM_WORKERS = NUM_CORES * NUM_SUBCORES
LANES = 16
CHUNK = 128
NT = 3


def _make_kernel(n_tok, L, D, V):
    n_chunks_w = n_tok // (NUM_WORKERS * CHUNK)
    DM = NT * D
    mesh = plsc.VectorSubcoreMesh(core_axis_name="c", subcore_axis_name="s")

    @functools.partial(
        pl.kernel,
        mesh=mesh,
        compiler_params=pltpu.CompilerParams(use_tc_tiling_on_sc=False),
        out_type=jax.ShapeDtypeStruct((n_tok, DM), jnp.float32),
        scratch_types=[
            pltpu.VMEM((NT, CHUNK), jnp.int32),
            pltpu.VMEM((NT, CHUNK), jnp.int32),
            pltpu.VMEM((NT, CHUNK, D), jnp.float32),
            pltpu.VMEM((NT, CHUNK, D), jnp.float32),
            pltpu.VMEM((L, DM), jnp.float32),
            pltpu.SemaphoreType.DMA,
            pltpu.SemaphoreType.DMA,
        ],
    )
    def emb_kernel(idxT, t0, t1, t2, pos, out,
                   iv0, iv1, r0, r1, pv, sem0, sem1):
        wid = lax.axis_index("s") * NUM_CORES + lax.axis_index("c")
        tables = (t0, t1, t2)
        pltpu.sync_copy(pos.at[pl.ds(0, L), pl.ds(0, DM)], pv)

        def load_idx(c, iv):
            base = (wid * n_chunks_w + c) * CHUNK
            for j in range(NT):
                pltpu.sync_copy(idxT.at[j, pl.ds(base, CHUNK)], iv.at[j])

        def fire_gathers(iv, r, sem):
            cps = []
            for j in range(NT):
                cps.append(pltpu.async_copy(tables[j].at[iv.at[j]],
                                            r.at[j], sem))
            return cps

        def wait_gathers(iv, r, sem):
            for j in range(NT):
                pltpu.make_async_copy(tables[j].at[iv.at[j]],
                                      r.at[j], sem).wait()

        def add_pos(c, r):
            base = (wid * n_chunks_w + c) * CHUNK
            pstart = lax.rem(base, L)

            def tok_body(t, carry):
                praw = pstart + t
                pr = lax.select(praw >= L, praw - L, praw)
                for j in range(NT):
                    for h in range(D // LANES):
                        x = pv[pr, pl.ds(j * D + h * LANES, LANES)]
                        plsc.addupdate(r.at[j, t, pl.ds(h * LANES, LANES)], x)
                return carry

            lax.fori_loop(0, CHUNK, tok_body, 0)

        def write_out(c, r):
            base = (wid * n_chunks_w + c) * CHUNK
            for j in range(NT):
                pltpu.sync_copy(r.at[j],
                                out.at[pl.ds(base, CHUNK), pl.ds(j * D, D)])

        bufs = ((iv0, r0, sem0), (iv1, r1, sem1))

        load_idx(0, iv0)
        fire_gathers(iv0, r0, sem0)
        load_idx(1, iv1)

        def pair_body(i, carry):
            for half in range(2):
                c = 2 * i + half
                iv_c, r_c, sem_c = bufs[half]
                iv_n, r_n, sem_n = bufs[1 - half]

                @pl.when(c + 1 < n_chunks_w)
                def _():
                    fire_gathers(iv_n, r_n, sem_n)

                wait_gathers(iv_c, r_c, sem_c)

                @pl.when(c + 2 < n_chunks_w)
                def _():
                    load_idx(c + 2, iv_c)

                add_pos(c, r_c)
                write_out(c, r_c)
            return carry

        lax.fori_loop(0, n_chunks_w // 2, pair_body, 0)

    return emb_kernel


def kernel(batch, W_opcode, W_operand1, W_operand2, W_pos):
    B, L, nt = batch.shape
    V, D = W_opcode.shape
    n_tok = B * L
    assert nt == NT and n_tok % (NUM_WORKERS * CHUNK) == 0
    assert (n_tok // (NUM_WORKERS * CHUNK)) % 2 == 0

    idxT = batch.astype(jnp.int32).reshape(n_tok, NT).T
    out = _make_kernel(n_tok, L, D, V)(
        idxT, W_opcode, W_operand1, W_operand2, W_pos)
    return out.reshape(B, L, NT * D)
